# baseline (speedup 1.0000x reference)
import functools

import jax
import jax.numpy as jnp
from jax import lax
from jax.experimental import pallas as pl
from jax.experimental.pallas import tpu as pltpu
from jax.experimental.pallas import tpu_sc as plsc

L = 16
NW = 32
K = 5
UNROLL = 4
CAND = UNROLL * K * L
GROUP = 2
B_SC = 32
LOG_K = 1.6094379124341003
NEG = float("-inf")


def _insert(stack, v):
    out = []
    new = v
    for m in stack:
        hi = jnp.maximum(m, new)
        new = jnp.minimum(m, new)
        out.append(hi)
    return out


def _make_sc_topcand(R, F, T):
    rows_per_w = R // NW
    slices = F // L
    n_groups = rows_per_w // GROUP
    out_cols = T * CAND
    w_out_rows = rows_per_w * CAND // out_cols
    mesh = plsc.VectorSubcoreMesh(core_axis_name="c", subcore_axis_name="s")

    @functools.partial(
        pl.kernel,
        mesh=mesh,
        out_type=jax.ShapeDtypeStruct((R * CAND // out_cols, out_cols), jnp.float32),
        scratch_types=[
            pltpu.VMEM((GROUP, F), jnp.float32),
            pltpu.VMEM((GROUP, F), jnp.float32),
            pltpu.VMEM((w_out_rows, out_cols), jnp.float32),
            pltpu.SemaphoreType.DMA,
            pltpu.SemaphoreType.DMA,
        ],
    )
    def sc_topcand(x_hbm, cand_hbm, buf_a, buf_b, outbuf, sem_a, sem_b):
        cid = lax.axis_index("c")
        sid = lax.axis_index("s")
        wid = cid * 16 + sid
        base_row = wid * rows_per_w

        def start_group(g, buf, sem):
            pltpu.async_copy(x_hbm.at[pl.ds(base_row + g * GROUP, GROUP), :], buf, sem)

        def wait_group(g, buf, sem):
            pltpu.make_async_copy(
                x_hbm.at[pl.ds(base_row + g * GROUP, GROUP), :], buf, sem
            ).wait()

        def compute_row(buf, j, r):
            def slice_body(i, accs):
                new_accs = []
                for u in range(UNROLL):
                    v = buf[j, pl.ds((i * UNROLL + u) * L, L)]
                    new_accs.append(tuple(_insert(list(accs[u]), v)))
                return tuple(new_accs)

            init = tuple(
                tuple(jnp.full((L,), NEG, jnp.float32) for _ in range(K))
                for _ in range(UNROLL)
            )
            accs = lax.fori_loop(0, slices // UNROLL, slice_body, init)
            q = 0
            for u in range(UNROLL):
                for m in accs[u]:
                    outbuf[r // T, pl.ds((r % T) * CAND + q * L, L)] = m
                    q += 1

        start_group(0, buf_a, sem_a)

        def outer_body(i, carry):
            ga = 2 * i
            gb = 2 * i + 1
            wait_group(ga, buf_a, sem_a)
            start_group(gb, buf_b, sem_b)
            for j in range(GROUP):
                compute_row(buf_a, j, ga * GROUP + j)
            wait_group(gb, buf_b, sem_b)
            ga_next = lax.rem(2 * i + 2, n_groups)
            start_group(ga_next, buf_a, sem_a)
            for j in range(GROUP):
                compute_row(buf_b, j, gb * GROUP + j)
            return carry

        lax.fori_loop(0, n_groups // 2, outer_body, 0)
        wait_group(0, buf_a, sem_a)

        pltpu.sync_copy(outbuf, cand_hbm.at[pl.ds(wid * w_out_rows, w_out_rows), :])

    return sc_topcand


def _tc_topcand_body(x_ref, out_ref):
    R, F = x_ref.shape
    n_chunks = F // 128
    stack = [jnp.full((R, 128), NEG, jnp.float32) for _ in range(K)]
    for c in range(n_chunks):
        stack = _insert(stack, x_ref[:, c * 128:(c + 1) * 128])

    ii = lax.broadcasted_iota(jnp.int32, (R, 128), 1)
    t1 = None
    se = jnp.zeros((R, 1), jnp.float32)
    ps = jnp.zeros((R, 1), jnp.float32)
    for i in range(K):
        mi = jnp.max(stack[0], axis=1, keepdims=True)
        for q in range(1, K):
            mi = jnp.maximum(mi, jnp.max(stack[q], axis=1, keepdims=True))
        if t1 is None:
            t1 = mi
        se = se + jnp.exp(mi - t1)
        ps = ps + 1.0 / (1.0 + jnp.exp(-mi))
        found = jnp.zeros((R, 1), jnp.bool_)
        for q in range(K):
            is_max = stack[q] == mi
            am = jnp.min(jnp.where(is_max, ii, 128), axis=1, keepdims=True)
            sel = (am < 128) & (~found)
            found = found | sel
            stack[q] = jnp.where(sel & (ii == am), NEG, stack[q])
    out_ref[:, 0:1] = jnp.log(se) + t1 - LOG_K
    out_ref[:, 1:2] = ps * (1.0 / K)


def _tower_stats(x, t1_only_width):
    W = t1_only_width
    ii = lax.broadcasted_iota(jnp.int32, x.shape, 1)
    t1 = jnp.max(x, axis=1, keepdims=True)
    se = jnp.zeros(t1.shape, jnp.float32)
    ps = jnp.zeros(t1.shape, jnp.float32)
    for i in range(K):
        mi = jnp.max(x, axis=1, keepdims=True)
        se = se + jnp.exp(mi - t1)
        ps = ps + 1.0 / (1.0 + jnp.exp(-mi))
        is_max = x == mi
        am = jnp.min(jnp.where(is_max, ii, W), axis=1, keepdims=True)
        x = jnp.where(ii == am, NEG, x)
    clip = jnp.log(se) + t1 - LOG_K
    pm = ps * (1.0 / K)
    return clip, pm


def _finalize_body(cand_sc_ref, stats_tc_ref, y_ref, out_ref):
    T = stats_tc_ref.shape[1] // 2
    y = y_ref[...]

    f_sc = None
    pms_sc = []
    for t in range(T):
        x = cand_sc_ref[:, t * CAND:(t + 1) * CAND]
        clip, pm = _tower_stats(x, CAND)
        f_sc = clip if f_sc is None else f_sc + clip
        pms_sc.append(pm)
    f_sc = f_sc * (1.0 / T)

    f_tc = None
    pms_tc = []
    for t in range(T):
        clip = stats_tc_ref[:, 2 * t:2 * t + 1]
        pm = stats_tc_ref[:, 2 * t + 1:2 * t + 2]
        f_tc = clip if f_tc is None else f_tc + clip
        pms_tc.append(pm)
    f_tc = f_tc * (1.0 / T)

    f = jnp.concatenate([f_sc, f_tc], axis=0)
    pms = [jnp.concatenate([a, b], axis=0)
           for a, b in zip(pms_sc, pms_tc)]

    sp = jnp.log1p(jnp.exp(-jnp.abs(f))) + jnp.maximum(f, 0.0)
    main = jnp.mean(y * (sp - f) + (1.0 - y) * sp)

    mu = sum(pms) * (1.0 / T)
    var = sum((pm - mu) ** 2 for pm in pms) * (1.0 / (T - 1))
    reg_cons = jnp.mean(jnp.sqrt(var)) * 0.1

    fp = 1.0 / (1.0 + jnp.exp(-f))
    pc = jnp.clip(fp, 1e-6, 1.0 - 1e-6)
    ent = -(pc * jnp.log(pc) + (1.0 - pc) * jnp.log(1.0 - pc))
    reg_ent = jnp.mean(ent) * 0.01

    out_ref[0, 0] = main + reg_cons + reg_ent


def kernel(tower_frame_logits, labels):
    B, T, F = tower_frame_logits.shape
    x = tower_frame_logits.reshape(B * T, F)
    b_sc = B_SC
    b_tc = B - b_sc

    cand_sc = _make_sc_topcand(b_sc * T, F, T)(x)

    BB = 8
    off = b_sc // BB
    stats_tc = pl.pallas_call(
        _tc_topcand_body,
        grid=(b_tc // BB,),
        in_specs=[pl.BlockSpec((BB * T, F), lambda i: (i + off, 0))],
        out_specs=pl.BlockSpec((BB * T, 2), lambda i: (i, 0)),
        out_shape=jax.ShapeDtypeStruct((b_tc * T, 2), jnp.float32),
    )(x)
    stats_tc = stats_tc.reshape(b_tc, 2 * T)

    y = labels.astype(jnp.float32).reshape(B, 1)

    loss = pl.pallas_call(
        _finalize_body,
        out_shape=jax.ShapeDtypeStruct((1, 1), jnp.float32),
        out_specs=pl.BlockSpec(memory_space=pltpu.SMEM),
    )(cand_sc, stats_tc, y)
    return loss[0, 0]


# speedup vs baseline: 1.2801x; 1.2801x over previous
import functools

import jax
import jax.numpy as jnp
from jax import lax
from jax.experimental import pallas as pl
from jax.experimental.pallas import tpu as pltpu
from jax.experimental.pallas import tpu_sc as plsc

L = 16
NW = 32
K = 5
UNROLL = 4
CAND = UNROLL * K * L
GROUP = 2
B_SC = 32
LOG_K = 1.6094379124341003
NEG = float("-inf")


def _insert(stack, v):
    out = []
    new = v
    for m in stack:
        hi = jnp.maximum(m, new)
        new = jnp.minimum(m, new)
        out.append(hi)
    return out


def _make_sc_topcand(R, F, T):
    rows_per_w = R // NW
    slices = F // L
    n_groups = rows_per_w // GROUP
    out_cols = T * CAND
    w_out_rows = rows_per_w * CAND // out_cols
    mesh = plsc.VectorSubcoreMesh(core_axis_name="c", subcore_axis_name="s")

    @functools.partial(
        pl.kernel,
        mesh=mesh,
        out_type=jax.ShapeDtypeStruct((R * CAND // out_cols, out_cols), jnp.float32),
        scratch_types=[
            pltpu.VMEM((GROUP, F), jnp.float32),
            pltpu.VMEM((GROUP, F), jnp.float32),
            pltpu.VMEM((w_out_rows, out_cols), jnp.float32),
            pltpu.SemaphoreType.DMA,
            pltpu.SemaphoreType.DMA,
        ],
    )
    def sc_topcand(x_hbm, cand_hbm, buf_a, buf_b, outbuf, sem_a, sem_b):
        cid = lax.axis_index("c")
        sid = lax.axis_index("s")
        wid = cid * 16 + sid
        base_row = wid * rows_per_w

        def start_group(g, buf, sem):
            pltpu.async_copy(x_hbm.at[pl.ds(base_row + g * GROUP, GROUP), :], buf, sem)

        def wait_group(g, buf, sem):
            pltpu.make_async_copy(
                x_hbm.at[pl.ds(base_row + g * GROUP, GROUP), :], buf, sem
            ).wait()

        def compute_row(buf, j, r):
            def slice_body(i, accs):
                new_accs = []
                for u in range(UNROLL):
                    v = buf[j, pl.ds((i * UNROLL + u) * L, L)]
                    new_accs.append(tuple(_insert(list(accs[u]), v)))
                return tuple(new_accs)

            init = tuple(
                tuple(jnp.full((L,), NEG, jnp.float32) for _ in range(K))
                for _ in range(UNROLL)
            )
            accs = lax.fori_loop(0, slices // UNROLL, slice_body, init)
            q = 0
            for u in range(UNROLL):
                for m in accs[u]:
                    outbuf[r // T, pl.ds((r % T) * CAND + q * L, L)] = m
                    q += 1

        start_group(0, buf_a, sem_a)

        def outer_body(i, carry):
            ga = 2 * i
            gb = 2 * i + 1
            wait_group(ga, buf_a, sem_a)
            start_group(gb, buf_b, sem_b)
            for j in range(GROUP):
                compute_row(buf_a, j, ga * GROUP + j)
            wait_group(gb, buf_b, sem_b)
            ga_next = lax.rem(2 * i + 2, n_groups)
            start_group(ga_next, buf_a, sem_a)
            for j in range(GROUP):
                compute_row(buf_b, j, gb * GROUP + j)
            return carry

        lax.fori_loop(0, n_groups // 2, outer_body, 0)
        wait_group(0, buf_a, sem_a)

        pltpu.sync_copy(outbuf, cand_hbm.at[pl.ds(wid * w_out_rows, w_out_rows), :])

    return sc_topcand


def _tc_topcand_body(x_ref, out_ref):
    R, F = x_ref.shape
    BB = out_ref.shape[0]
    T = R // BB
    n_chunks = F // 128
    stack = [jnp.full((R, 128), NEG, jnp.float32) for _ in range(K)]
    for c in range(n_chunks):
        stack = _insert(stack, x_ref[:, c * 128:(c + 1) * 128])
    for b in range(BB):
        for t in range(T):
            for q in range(K):
                r = b * T + t
                out_ref[b:b + 1, pl.ds((t * K + q) * 128, 128)] = stack[q][r:r + 1, :]


def _tower_stats(x, t1_only_width):
    W = t1_only_width
    ii = lax.broadcasted_iota(jnp.int32, x.shape, 1)
    t1 = jnp.max(x, axis=1, keepdims=True)
    se = jnp.zeros(t1.shape, jnp.float32)
    ps = jnp.zeros(t1.shape, jnp.float32)
    for i in range(K):
        mi = jnp.max(x, axis=1, keepdims=True)
        se = se + jnp.exp(mi - t1)
        ps = ps + 1.0 / (1.0 + jnp.exp(-mi))
        is_max = x == mi
        am = jnp.min(jnp.where(is_max, ii, W), axis=1, keepdims=True)
        x = jnp.where(ii == am, NEG, x)
    clip = jnp.log(se) + t1 - LOG_K
    pm = ps * (1.0 / K)
    return clip, pm


def _finalize_body(cand_sc_ref, cand_tc_ref, y_ref, out_ref):
    Bs = cand_sc_ref.shape[0]
    T = cand_sc_ref.shape[1] // CAND
    Wt = cand_tc_ref.shape[1] // T
    y = y_ref[...]
    B = y.shape[0]

    f_parts, pm_parts = [], []
    for part, (ref, W) in enumerate([(cand_sc_ref, CAND), (cand_tc_ref, Wt)]):
        f_sum = None
        pms = []
        for t in range(T):
            x = ref[:, t * W:(t + 1) * W]
            clip, pm = _tower_stats(x, W)
            f_sum = clip if f_sum is None else f_sum + clip
            pms.append(pm)
        f_parts.append(f_sum * (1.0 / T))
        pm_parts.append(pms)

    f = jnp.concatenate([f_parts[0], f_parts[1]], axis=0)
    pms = [jnp.concatenate([a, b], axis=0)
           for a, b in zip(pm_parts[0], pm_parts[1])]

    sp = jnp.log1p(jnp.exp(-jnp.abs(f))) + jnp.maximum(f, 0.0)
    main = jnp.mean(y * (sp - f) + (1.0 - y) * sp)

    mu = sum(pms) * (1.0 / T)
    var = sum((pm - mu) ** 2 for pm in pms) * (1.0 / (T - 1))
    reg_cons = jnp.mean(jnp.sqrt(var)) * 0.1

    fp = 1.0 / (1.0 + jnp.exp(-f))
    pc = jnp.clip(fp, 1e-6, 1.0 - 1e-6)
    ent = -(pc * jnp.log(pc) + (1.0 - pc) * jnp.log(1.0 - pc))
    reg_ent = jnp.mean(ent) * 0.01

    out_ref[0, 0] = main + reg_cons + reg_ent


def kernel(tower_frame_logits, labels):
    B, T, F = tower_frame_logits.shape
    x = tower_frame_logits.reshape(B * T, F)
    b_sc = B_SC
    b_tc = B - b_sc

    cand_sc = _make_sc_topcand(b_sc * T, F, T)(x)

    BB = 16
    off = b_sc // BB
    cand_tc = pl.pallas_call(
        _tc_topcand_body,
        grid=(b_tc // BB,),
        in_specs=[pl.BlockSpec((BB * T, F), lambda i: (i + off, 0))],
        out_specs=pl.BlockSpec((BB, T * K * 128), lambda i: (i, 0)),
        out_shape=jax.ShapeDtypeStruct((b_tc, T * K * 128), jnp.float32),
    )(x)

    y = labels.astype(jnp.float32).reshape(B, 1)

    loss = pl.pallas_call(
        _finalize_body,
        out_shape=jax.ShapeDtypeStruct((1, 1), jnp.float32),
        out_specs=pl.BlockSpec(memory_space=pltpu.SMEM),
    )(cand_sc, cand_tc, y)
    return loss[0, 0]


# speedup vs baseline: 1.2887x; 1.0067x over previous
import functools

import jax
import jax.numpy as jnp
from jax import lax
from jax.experimental import pallas as pl
from jax.experimental.pallas import tpu as pltpu
from jax.experimental.pallas import tpu_sc as plsc

L = 16
NW = 32
K = 5
UNROLL = 4
CAND = UNROLL * K * L
GROUP = 2
B_SC = 32
LOG_K = 1.6094379124341003
NEG = float("-inf")


def _insert(stack, v):
    out = []
    new = v
    for m in stack:
        hi = jnp.maximum(m, new)
        new = jnp.minimum(m, new)
        out.append(hi)
    return out


def _make_sc_topcand(R, F, T):
    rows_per_w = R // NW
    slices = F // L
    n_groups = rows_per_w // GROUP
    out_cols = T * CAND
    w_out_rows = rows_per_w * CAND // out_cols
    mesh = plsc.VectorSubcoreMesh(core_axis_name="c", subcore_axis_name="s")

    @functools.partial(
        pl.kernel,
        mesh=mesh,
        out_type=jax.ShapeDtypeStruct((R * CAND // out_cols, out_cols), jnp.float32),
        scratch_types=[
            pltpu.VMEM((GROUP, F), jnp.float32),
            pltpu.VMEM((GROUP, F), jnp.float32),
            pltpu.VMEM((w_out_rows, out_cols), jnp.float32),
            pltpu.SemaphoreType.DMA,
            pltpu.SemaphoreType.DMA,
        ],
    )
    def sc_topcand(x_hbm, cand_hbm, buf_a, buf_b, outbuf, sem_a, sem_b):
        cid = lax.axis_index("c")
        sid = lax.axis_index("s")
        wid = cid * 16 + sid
        base_row = wid * rows_per_w

        def start_group(g, buf, sem):
            pltpu.async_copy(x_hbm.at[pl.ds(base_row + g * GROUP, GROUP), :], buf, sem)

        def wait_group(g, buf, sem):
            pltpu.make_async_copy(
                x_hbm.at[pl.ds(base_row + g * GROUP, GROUP), :], buf, sem
            ).wait()

        def compute_row(buf, j, r):
            def slice_body(i, accs):
                new_accs = []
                for u in range(UNROLL):
                    v = buf[j, pl.ds((i * UNROLL + u) * L, L)]
                    new_accs.append(tuple(_insert(list(accs[u]), v)))
                return tuple(new_accs)

            init = tuple(
                tuple(jnp.full((L,), NEG, jnp.float32) for _ in range(K))
                for _ in range(UNROLL)
            )
            accs = lax.fori_loop(0, slices // UNROLL, slice_body, init)
            q = 0
            for u in range(UNROLL):
                for m in accs[u]:
                    outbuf[r // T, pl.ds((r % T) * CAND + q * L, L)] = m
                    q += 1

        start_group(0, buf_a, sem_a)

        def outer_body(i, carry):
            ga = 2 * i
            gb = 2 * i + 1
            wait_group(ga, buf_a, sem_a)
            start_group(gb, buf_b, sem_b)
            for j in range(GROUP):
                compute_row(buf_a, j, ga * GROUP + j)
            wait_group(gb, buf_b, sem_b)
            ga_next = lax.rem(2 * i + 2, n_groups)
            start_group(ga_next, buf_a, sem_a)
            for j in range(GROUP):
                compute_row(buf_b, j, gb * GROUP + j)
            return carry

        lax.fori_loop(0, n_groups // 2, outer_body, 0)
        wait_group(0, buf_a, sem_a)

        pltpu.sync_copy(outbuf, cand_hbm.at[pl.ds(wid * w_out_rows, w_out_rows), :])

    return sc_topcand


def _tc_topcand_body(x_ref, out_ref):
    R, F = x_ref.shape
    BB = out_ref.shape[0]
    T = R // BB
    n_chunks = F // 128
    stack = [jnp.full((R, 128), NEG, jnp.float32) for _ in range(K)]
    for c in range(n_chunks):
        stack = _insert(stack, x_ref[:, c * 128:(c + 1) * 128])
    for b in range(BB):
        for t in range(T):
            for q in range(K):
                r = b * T + t
                out_ref[b:b + 1, pl.ds((t * K + q) * 128, 128)] = stack[q][r:r + 1, :]


def _tower_stats(x, t1_only_width):
    W = t1_only_width
    ii = lax.broadcasted_iota(jnp.int32, x.shape, 1)
    t1 = jnp.max(x, axis=1, keepdims=True)
    se = jnp.zeros(t1.shape, jnp.float32)
    ps = jnp.zeros(t1.shape, jnp.float32)
    for i in range(K):
        mi = jnp.max(x, axis=1, keepdims=True)
        se = se + jnp.exp(mi - t1)
        ps = ps + 1.0 / (1.0 + jnp.exp(-mi))
        is_max = x == mi
        am = jnp.min(jnp.where(is_max, ii, W), axis=1, keepdims=True)
        x = jnp.where(ii == am, NEG, x)
    clip = jnp.log(se) + t1 - LOG_K
    pm = ps * (1.0 / K)
    return clip, pm


def _finalize_body(cand_sc_ref, cand_tc_ref, y_ref, out_ref):
    Bs = cand_sc_ref.shape[0]
    T = cand_sc_ref.shape[1] // CAND
    Wt = cand_tc_ref.shape[1] // T
    y = y_ref[...]
    B = y.shape[0]

    f_parts, pm_parts = [], []
    for part, (ref, W) in enumerate([(cand_sc_ref, CAND), (cand_tc_ref, Wt)]):
        f_sum = None
        pms = []
        for t in range(T):
            x = ref[:, t * W:(t + 1) * W]
            clip, pm = _tower_stats(x, W)
            f_sum = clip if f_sum is None else f_sum + clip
            pms.append(pm)
        f_parts.append(f_sum * (1.0 / T))
        pm_parts.append(pms)

    f = jnp.concatenate([f_parts[0], f_parts[1]], axis=0)
    pms = [jnp.concatenate([a, b], axis=0)
           for a, b in zip(pm_parts[0], pm_parts[1])]

    sp = jnp.log1p(jnp.exp(-jnp.abs(f))) + jnp.maximum(f, 0.0)
    main = jnp.mean(y * (sp - f) + (1.0 - y) * sp)

    mu = sum(pms) * (1.0 / T)
    var = sum((pm - mu) ** 2 for pm in pms) * (1.0 / (T - 1))
    reg_cons = jnp.mean(jnp.sqrt(var)) * 0.1

    fp = 1.0 / (1.0 + jnp.exp(-f))
    pc = jnp.clip(fp, 1e-6, 1.0 - 1e-6)
    ent = -(pc * jnp.log(pc) + (1.0 - pc) * jnp.log(1.0 - pc))
    reg_ent = jnp.mean(ent) * 0.01

    out_ref[0, 0] = main + reg_cons + reg_ent


def kernel(tower_frame_logits, labels):
    B, T, F = tower_frame_logits.shape
    x = tower_frame_logits.reshape(B * T, F)
    b_sc = B_SC
    b_tc = B - b_sc

    cand_sc = _make_sc_topcand(b_sc * T, F, T)(x)

    BB = 32
    off = b_sc // BB
    cand_tc = pl.pallas_call(
        _tc_topcand_body,
        grid=(b_tc // BB,),
        in_specs=[pl.BlockSpec((BB * T, F), lambda i: (i + off, 0))],
        out_specs=pl.BlockSpec((BB, T * K * 128), lambda i: (i, 0)),
        out_shape=jax.ShapeDtypeStruct((b_tc, T * K * 128), jnp.float32),
    )(x)

    y = labels.astype(jnp.float32).reshape(B, 1)

    loss = pl.pallas_call(
        _finalize_body,
        out_shape=jax.ShapeDtypeStruct((1, 1), jnp.float32),
        out_specs=pl.BlockSpec(memory_space=pltpu.SMEM),
    )(cand_sc, cand_tc, y)
    return loss[0, 0]
